# baseline (speedup 1.0000x reference)
import functools

import jax
import jax.numpy as jnp
from jax import lax
from jax.experimental import pallas as pl
from jax.experimental.pallas import tpu as pltpu
from jax.experimental.pallas import tpu_sc as plsc

B, N, K = 2, 1024, 16
EPS = 1e-5
NEG_INF = float(jnp.finfo(jnp.float32).min)

_SC_CORES = 2
_SC_SUBCORES = 16
_NW = _SC_CORES * _SC_SUBCORES


def _leaky(x):
    return jnp.where(x >= 0, x, 0.2 * x)


def _dot_t(x, w):
    return lax.dot_general(
        x, w, dimension_numbers=(((1,), (1,)), ((), ())),
        preferred_element_type=jnp.float32,
        precision=lax.Precision.HIGHEST)


def _knn_body(c_ref, ct_ref, idx_ref, cnt_ref):
    b = pl.program_id(0)
    c = c_ref[0]
    ct = ct_ref[0]
    q0, q1, q2 = c[:, 0:1], c[:, 1:2], c[:, 2:3]
    k0, k1, k2 = ct[0:1, :], ct[1:2, :], ct[2:3, :]
    nq = q0 * q0 + q1 * q1 + q2 * q2
    nk = k0 * k0 + k1 * k1 + k2 * k2
    rb = lambda x: x.astype(jnp.bfloat16).astype(jnp.float32)
    dot = (rb(q0) * rb(k0) + rb(q1) * rb(k1)) + rb(q2) * rb(k2)
    nd = 2.0 * dot - nq - nk
    mi = lax.broadcasted_iota(jnp.int32, (N, N), 1)
    cnt = jnp.zeros((1, N), jnp.float32)
    cols = []
    for _ in range(K):
        mval = jnp.max(nd, axis=1, keepdims=True)
        cand = jnp.where(nd == mval, mi, N)
        arg = jnp.min(cand, axis=1, keepdims=True)
        hit = mi == arg
        cnt = cnt + jnp.sum(hit.astype(jnp.float32), axis=0, keepdims=True)
        nd = jnp.where(hit, NEG_INF, nd)
        cols.append(arg)
    idx = jnp.concatenate(cols, axis=1) + b * N
    idx_ref[0] = idx
    cnt_ref[0] = cnt


def _knn(coor):
    coor_t = jnp.transpose(coor, (0, 2, 1))
    return pl.pallas_call(
        _knn_body,
        grid=(B,),
        in_specs=[
            pl.BlockSpec((1, N, 3), lambda b: (b, 0, 0)),
            pl.BlockSpec((1, 3, N), lambda b: (b, 0, 0)),
        ],
        out_specs=[
            pl.BlockSpec((1, N, K), lambda b: (b, 0, 0)),
            pl.BlockSpec((1, 1, N), lambda b: (b, 0, 0)),
        ],
        out_shape=[
            jax.ShapeDtypeStruct((B, N, K), jnp.int32),
            jax.ShapeDtypeStruct((B, 1, N), jnp.float32),
        ],
    )(coor, coor_t)


def _intrans_body(f_ref, w_ref, b_ref, o_ref):
    o_ref[0] = _dot_t(f_ref[0], w_ref[...]) + b_ref[...]


def _intrans(f, w_in, b_in):
    co, ci = w_in.shape
    return pl.pallas_call(
        _intrans_body,
        grid=(B,),
        in_specs=[
            pl.BlockSpec((1, N, ci), lambda b: (b, 0, 0)),
            pl.BlockSpec((co, ci), lambda b: (0, 0)),
            pl.BlockSpec((1, co), lambda b: (0, 0)),
        ],
        out_specs=pl.BlockSpec((1, N, co), lambda b: (b, 0, 0)),
        out_shape=jax.ShapeDtypeStruct((B, N, co), jnp.float32),
    )(f, w_in, b_in.reshape(1, co))


def _mm_body(ci, f_ref, w_ref, a_ref, abf_ref, d_ref):
    fb = f_ref[0]
    wa = w_ref[:, :ci]
    wd = w_ref[:, ci:] - wa
    a = _dot_t(fb, wa)
    a_ref[0] = a
    abf_ref[0] = a.astype(jnp.bfloat16)
    d_ref[0] = _dot_t(fb, wd)


def _mm(f, w):
    co = w.shape[0]
    ci = w.shape[1] // 2
    return pl.pallas_call(
        functools.partial(_mm_body, ci),
        grid=(B,),
        in_specs=[
            pl.BlockSpec((1, N, ci), lambda b: (b, 0, 0)),
            pl.BlockSpec((co, 2 * ci), lambda b: (0, 0)),
        ],
        out_specs=[
            pl.BlockSpec((1, N, co), lambda b: (b, 0, 0)),
            pl.BlockSpec((1, N, co), lambda b: (b, 0, 0)),
            pl.BlockSpec((1, N, co), lambda b: (b, 0, 0)),
        ],
        out_shape=[
            jax.ShapeDtypeStruct((B, N, co), jnp.float32),
            jax.ShapeDtypeStruct((B, N, co), jnp.bfloat16),
            jax.ShapeDtypeStruct((B, N, co), jnp.float32),
        ],
    )(f, w)


@functools.lru_cache(maxsize=None)
def _make_sc_gather_reduce(c):
    bn = B * N
    tw = bn // _NW
    cw = c // 2
    c32 = c // 32
    ob = 8
    mesh = plsc.VectorSubcoreMesh(core_axis_name="c", subcore_axis_name="s")

    @functools.partial(
        pl.kernel,
        out_type=(
            jax.ShapeDtypeStruct((bn, cw), jnp.float32),
            jax.ShapeDtypeStruct((bn, cw), jnp.float32),
        ),
        mesh=mesh,
        compiler_params=pltpu.CompilerParams(needs_layout_passes=False),
        scratch_types=[
            pltpu.VMEM((tw, K), jnp.int32),
            pltpu.VMEM((K, cw), jnp.float32),
            pltpu.VMEM((K, cw), jnp.float32),
            pltpu.VMEM((2 * ob, cw), jnp.float32),
            pltpu.VMEM((2 * ob, cw), jnp.float32),
            pltpu.SemaphoreType.DMA,
            pltpu.SemaphoreType.DMA,
            pltpu.SemaphoreType.DMA,
            pltpu.SemaphoreType.DMA,
        ],
    )
    def sck(a_hbm, idx_hbm, smax_hbm, ssum_hbm,
            idxv, rows0, rows1, omax, osum, sem0, sem1, osem0, osem1):
        wid = lax.axis_index("s") * _SC_CORES + lax.axis_index("c")
        base = wid * tw
        pltpu.sync_copy(idx_hbm.at[pl.ds(base, tw)], idxv)

        rows = (rows0, rows1)
        sems = (sem0, sem1)
        osems = (osem0, osem1)

        def start(t, slot):
            return pltpu.async_copy(a_hbm.at[idxv.at[t]], rows[slot], sems[slot])

        def wait(t, slot):
            pltpu.make_async_copy(
                a_hbm.at[idxv.at[t]], rows[slot], sems[slot]).wait()

        def reduce_task(slot, obuf, orow):
            buf = rows[slot]

            def chunk(j, _):
                sl = pl.ds(j * 16, 16)
                v = [plsc.unpack(plsc.bitcast(buf[r, sl], jnp.bfloat16),
                                 format=plsc.PackFormat.INTERLEAVED)
                     for r in range(K)]
                ma, mb = v[0]
                sa, sb = v[0]
                for r in range(1, K):
                    va, vb = v[r]
                    ma = jnp.maximum(ma, va)
                    mb = jnp.maximum(mb, vb)
                    sa = sa + va
                    sb = sb + vb
                mp = plsc.pack(ma, mb, format=plsc.PackFormat.INTERLEAVED)
                sp = plsc.pack(sa, sb, format=plsc.PackFormat.INTERLEAVED)
                omax[obuf * ob + orow, sl] = plsc.bitcast(mp, jnp.float32)
                osum[obuf * ob + orow, sl] = plsc.bitcast(sp, jnp.float32)
                return 0

            lax.fori_loop(0, c32, chunk, 0, unroll=2)

        def out_copies(blk, obuf):
            orow0 = base + blk * ob
            osl = pl.ds(obuf * ob, ob)
            return (
                pltpu.make_async_copy(
                    omax.at[osl], smax_hbm.at[pl.ds(orow0, ob)], osems[obuf]),
                pltpu.make_async_copy(
                    osum.at[osl], ssum_hbm.at[pl.ds(orow0, ob)], osems[obuf]),
            )

        start(0, 0)
        start(1, 1)

        def block_pair(p, _):
            for q in range(2):
                blk = 2 * p + q
                t0 = blk * ob

                @pl.when(blk >= 2)
                def _():
                    for cp in out_copies(blk - 2, q):
                        cp.wait()

                for i in range(ob):
                    t = t0 + i
                    slot = i % 2
                    wait(t, slot)
                    reduce_task(slot, q, i)

                    @pl.when(t + 2 < tw)
                    def _():
                        start(t + 2, slot)

                for cp in out_copies(blk, q):
                    cp.start()
            return 0

        nblk = tw // ob
        lax.fori_loop(0, nblk // 2, block_pair, 0)
        for cp in out_copies(nblk - 2, 0):
            cp.wait()
        for cp in out_copies(nblk - 1, 1):
            cp.wait()

    return sck


def _sc_gather_reduce(abf, gidx2):
    c = abf.shape[-1]
    ap = lax.bitcast_convert_type(
        abf.reshape(B * N, c // 2, 2), jnp.float32)
    smax_p, ssum_p = _make_sc_gather_reduce(c)(ap, gidx2)

    def unpack(x):
        return lax.bitcast_convert_type(x, jnp.bfloat16).reshape(B, N, c)

    return unpack(smax_p), unpack(ssum_p)


def _stats_body(c, a_ref, d_ref, mx_ref, sm_ref, cnt_ref, gw_ref, gb_ref, o_ref):
    cg = c // 4
    a = a_ref[0]
    d = d_ref[0]
    mx = mx_ref[0].astype(jnp.float32)
    sm = sm_ref[0].astype(jnp.float32)
    cnt = cnt_ref[0]
    col_a = jnp.sum(a * cnt, axis=0, keepdims=True)
    col_a2 = jnp.sum(a * a * cnt, axis=0, keepdims=True)
    col_d = jnp.sum(d, axis=0, keepdims=True)
    col_d2 = jnp.sum(d * d, axis=0, keepdims=True)
    col_x = jnp.sum(d * sm, axis=0, keepdims=True)
    s1 = col_a + K * col_d
    s2 = col_a2 + 2.0 * col_x + K * col_d2
    ci = lax.broadcasted_iota(jnp.int32, (1, c), 1)
    denom = float(cg * N * K)
    mean_bc = jnp.zeros((1, c), jnp.float32)
    inv_bc = jnp.zeros((1, c), jnp.float32)
    for g in range(4):
        msk = (ci >= g * cg) & (ci < (g + 1) * cg)
        zero = jnp.zeros((1, c), jnp.float32)
        s1g = jnp.sum(jnp.where(msk, s1, zero), axis=1, keepdims=True)
        s2g = jnp.sum(jnp.where(msk, s2, zero), axis=1, keepdims=True)
        mean_g = s1g / denom
        var_g = s2g / denom - mean_g * mean_g
        inv_g = lax.rsqrt(var_g + EPS)
        mean_bc = jnp.where(msk, mean_bc + mean_g, mean_bc)
        inv_bc = jnp.where(msk, inv_bc + inv_g, inv_bc)
    xn = (d + mx - mean_bc) * inv_bc * gw_ref[...] + gb_ref[...]
    o_ref[0] = _leaky(xn)


def _stats(a, d, smax, ssum, cnt3, gnw, gnb):
    c = a.shape[-1]
    spec = pl.BlockSpec((1, N, c), lambda b: (b, 0, 0))
    wspec = pl.BlockSpec((1, c), lambda b: (0, 0))
    return pl.pallas_call(
        functools.partial(_stats_body, c),
        grid=(B,),
        in_specs=[spec, spec, spec, spec,
                  pl.BlockSpec((1, N, 1), lambda b: (b, 0, 0)),
                  wspec, wspec],
        out_specs=spec,
        out_shape=jax.ShapeDtypeStruct((B, N, c), jnp.float32),
    )(a, d, smax, ssum, cnt3, gnw.reshape(1, c), gnb.reshape(1, c))


def _final_body(f1_ref, f2_ref, f3_ref, f4_ref, w_ref, gw_ref, gb_ref, o_ref):
    outc = w_ref.shape[0]
    cg = outc // 4
    p = _dot_t(f1_ref[0], w_ref[:, 0:256])
    p = p + _dot_t(f2_ref[0], w_ref[:, 256:768])
    p = p + _dot_t(f3_ref[0], w_ref[:, 768:1280])
    p = p + _dot_t(f4_ref[0], w_ref[:, 1280:2304])
    s1 = jnp.sum(p, axis=0, keepdims=True)
    s2 = jnp.sum(p * p, axis=0, keepdims=True)
    ci = lax.broadcasted_iota(jnp.int32, (1, outc), 1)
    denom = float(cg * N)
    mean_bc = jnp.zeros((1, outc), jnp.float32)
    inv_bc = jnp.zeros((1, outc), jnp.float32)
    for g in range(4):
        msk = (ci >= g * cg) & (ci < (g + 1) * cg)
        zero = jnp.zeros((1, outc), jnp.float32)
        s1g = jnp.sum(jnp.where(msk, s1, zero), axis=1, keepdims=True)
        s2g = jnp.sum(jnp.where(msk, s2, zero), axis=1, keepdims=True)
        mean_g = s1g / denom
        var_g = s2g / denom - mean_g * mean_g
        inv_g = lax.rsqrt(var_g + EPS)
        mean_bc = jnp.where(msk, mean_bc + mean_g, mean_bc)
        inv_bc = jnp.where(msk, inv_bc + inv_g, inv_bc)
    xn = (p - mean_bc) * inv_bc * gw_ref[...] + gb_ref[...]
    o_ref[0] = _leaky(xn)


def _final(f1, f2, f3, f4, w5, gw, gb):
    outc = w5.shape[0]
    return pl.pallas_call(
        _final_body,
        grid=(B,),
        in_specs=[
            pl.BlockSpec((1, N, 256), lambda b: (b, 0, 0)),
            pl.BlockSpec((1, N, 512), lambda b: (b, 0, 0)),
            pl.BlockSpec((1, N, 512), lambda b: (b, 0, 0)),
            pl.BlockSpec((1, N, 1024), lambda b: (b, 0, 0)),
            pl.BlockSpec((outc, 2304), lambda b: (0, 0)),
            pl.BlockSpec((1, outc), lambda b: (0, 0)),
            pl.BlockSpec((1, outc), lambda b: (0, 0)),
        ],
        out_specs=pl.BlockSpec((1, N, outc), lambda b: (b, 0, 0)),
        out_shape=jax.ShapeDtypeStruct((B, N, outc), jnp.float32),
    )(f1, f2, f3, f4, w5, gw.reshape(1, outc), gb.reshape(1, outc))


def _layer(f, w, gnw, gnb, gidx2, cnt3):
    a, abf, d = _mm(f, w)
    smax, ssum = _sc_gather_reduce(abf, gidx2)
    return _stats(a, d, smax, ssum, cnt3, gnw, gnb)


def kernel(f, coor, W_in, b_in, W1, gn1_w, gn1_b, W2, gn2_w, gn2_b,
           W3, gn3_w, gn3_b, W4, gn4_w, gn4_b, W5, gn5_w, gn5_b):
    gidx, cnt = _knn(coor)
    gidx2 = gidx.reshape(B * N, K)
    cnt3 = jnp.transpose(cnt, (0, 2, 1))
    f0 = _intrans(f, W_in, b_in)
    f1 = _layer(f0, W1, gn1_w, gn1_b, gidx2, cnt3)
    f2 = _layer(f1, W2, gn2_w, gn2_b, gidx2, cnt3)
    f3 = _layer(f2, W3, gn3_w, gn3_b, gidx2, cnt3)
    f4 = _layer(f3, W4, gn4_w, gn4_b, gidx2, cnt3)
    return _final(f1, f2, f3, f4, W5, gn5_w, gn5_b)


# speedup vs baseline: 1.5396x; 1.5396x over previous
import functools

import jax
import jax.numpy as jnp
from jax import lax
from jax.experimental import pallas as pl
from jax.experimental.pallas import tpu as pltpu
from jax.experimental.pallas import tpu_sc as plsc

B, N, K = 2, 1024, 16
EPS = 1e-5
NEG_INF = float(jnp.finfo(jnp.float32).min)

_SC_CORES = 2
_SC_SUBCORES = 16
_NW = _SC_CORES * _SC_SUBCORES


def _leaky(x):
    return jnp.where(x >= 0, x, 0.2 * x)


def _dot_t(x, w):
    return lax.dot_general(
        x, w, dimension_numbers=(((1,), (1,)), ((), ())),
        preferred_element_type=jnp.float32,
        precision=lax.Precision.HIGHEST)


def _knn_body(c_ref, ct_ref, idx_ref, cnt_ref):
    c = c_ref[0]
    ct = ct_ref[0]
    q0, q1, q2 = c[:, 0:1], c[:, 1:2], c[:, 2:3]
    k0, k1, k2 = ct[0:1, :], ct[1:2, :], ct[2:3, :]
    nq = q0 * q0 + q1 * q1 + q2 * q2
    nk = k0 * k0 + k1 * k1 + k2 * k2
    rb = lambda x: x.astype(jnp.bfloat16).astype(jnp.float32)
    dot = (rb(q0) * rb(k0) + rb(q1) * rb(k1)) + rb(q2) * rb(k2)
    nd = 2.0 * dot - nq - nk
    mi = lax.broadcasted_iota(jnp.int32, (N, N), 1)
    cnt = jnp.zeros((1, N), jnp.float32)
    cols = []
    for _ in range(K):
        mval = jnp.max(nd, axis=1, keepdims=True)
        cand = jnp.where(nd == mval, mi, N)
        arg = jnp.min(cand, axis=1, keepdims=True)
        hit = mi == arg
        cnt = cnt + jnp.sum(hit.astype(jnp.float32), axis=0, keepdims=True)
        nd = jnp.where(hit, NEG_INF, nd)
        cols.append(arg)
    idx_ref[0] = jnp.concatenate(cols, axis=1)

    cnt_ref[0] = cnt


def _knn(coor):
    coor_t = jnp.transpose(coor, (0, 2, 1))
    return pl.pallas_call(
        _knn_body,
        grid=(B,),
        in_specs=[
            pl.BlockSpec((1, N, 3), lambda b: (b, 0, 0)),
            pl.BlockSpec((1, 3, N), lambda b: (b, 0, 0)),
        ],
        out_specs=[
            pl.BlockSpec((1, N, K), lambda b: (b, 0, 0)),
            pl.BlockSpec((1, 1, N), lambda b: (b, 0, 0)),
        ],
        out_shape=[
            jax.ShapeDtypeStruct((B, N, K), jnp.int32),
            jax.ShapeDtypeStruct((B, 1, N), jnp.float32),
        ],
    )(coor, coor_t)


def _intrans_body(f_ref, w_ref, b_ref, o_ref):
    o_ref[0] = _dot_t(f_ref[0], w_ref[...]) + b_ref[...]


def _intrans(f, w_in, b_in):
    co, ci = w_in.shape
    nb = f.shape[0]
    return pl.pallas_call(
        _intrans_body,
        grid=(nb,),
        in_specs=[
            pl.BlockSpec((1, N, ci), lambda b: (b, 0, 0)),
            pl.BlockSpec((co, ci), lambda b: (0, 0)),
            pl.BlockSpec((1, co), lambda b: (0, 0)),
        ],
        out_specs=pl.BlockSpec((1, N, co), lambda b: (b, 0, 0)),
        out_shape=jax.ShapeDtypeStruct((B, N, co), jnp.float32),
    )(f, w_in, b_in.reshape(1, co))


def _mm_body(ci, f_ref, w_ref, a_ref, d_ref):
    fb = f_ref[0]
    wa = w_ref[:, :ci]
    wd = w_ref[:, ci:] - wa
    a_ref[0] = _dot_t(fb, wa)
    d_ref[0] = _dot_t(fb, wd)


def _mm(f, w):
    co = w.shape[0]
    ci = w.shape[1] // 2
    nb = f.shape[0]
    return pl.pallas_call(
        functools.partial(_mm_body, ci),
        grid=(nb,),
        in_specs=[
            pl.BlockSpec((1, N, ci), lambda b: (b, 0, 0)),
            pl.BlockSpec((co, 2 * ci), lambda b: (0, 0)),
        ],
        out_specs=[
            pl.BlockSpec((1, N, co), lambda b: (b, 0, 0)),
            pl.BlockSpec((1, N, co), lambda b: (b, 0, 0)),
        ],
        out_shape=[
            jax.ShapeDtypeStruct((nb, N, co), jnp.float32),
            jax.ShapeDtypeStruct((nb, N, co), jnp.float32),
        ],
    )(f, w)


@functools.lru_cache(maxsize=None)
def _make_sc_gather_reduce(c, bn):
    tw = bn // _NW
    c16 = c // 16
    ob = 8
    mesh = plsc.VectorSubcoreMesh(core_axis_name="c", subcore_axis_name="s")

    @functools.partial(
        pl.kernel,
        out_type=(
            jax.ShapeDtypeStruct((bn, c), jnp.float32),
            jax.ShapeDtypeStruct((bn, c), jnp.float32),
        ),
        mesh=mesh,
        scratch_types=[
            pltpu.VMEM((tw, K), jnp.int32),
            pltpu.VMEM((K, c), jnp.float32),
            pltpu.VMEM((K, c), jnp.float32),
            pltpu.VMEM((ob, c), jnp.float32),
            pltpu.VMEM((ob, c), jnp.float32),
            pltpu.SemaphoreType.DMA,
            pltpu.SemaphoreType.DMA,
        ],
    )
    def sck(a_hbm, idx_hbm, smax_hbm, ssum_hbm,
            idxv, rows0, rows1, omax, osum, sem0, sem1):
        wid = lax.axis_index("s") * _SC_CORES + lax.axis_index("c")
        base = wid * tw
        pltpu.sync_copy(idx_hbm.at[pl.ds(base, tw)], idxv)

        rows = (rows0, rows1)
        sems = (sem0, sem1)

        def start(t, slot):
            return pltpu.async_copy(a_hbm.at[idxv.at[t]], rows[slot], sems[slot])

        def wait(t, slot):
            pltpu.make_async_copy(
                a_hbm.at[idxv.at[t]], rows[slot], sems[slot]).wait()

        def reduce_task(slot, orow):
            buf = rows[slot]

            def chunk(j, _):
                sl = pl.ds(j * 16, 16)
                m = buf[0, sl]
                s = m
                for r in range(1, K):
                    v = buf[r, sl]
                    m = jnp.maximum(m, v)
                    s = s + v
                omax[orow, sl] = m
                osum[orow, sl] = s
                return 0

            lax.fori_loop(0, c16, chunk, 0, unroll=2)

        start(0, 0)
        start(1, 1)

        def block(blk, _):
            t0 = blk * ob
            for i in range(ob):
                t = t0 + i
                slot = i % 2
                wait(t, slot)
                reduce_task(slot, i)

                @pl.when(t + 2 < tw)
                def _():
                    start(t + 2, slot)

            orow0 = base + t0
            pltpu.sync_copy(omax, smax_hbm.at[pl.ds(orow0, ob)])
            pltpu.sync_copy(osum, ssum_hbm.at[pl.ds(orow0, ob)])
            return 0

        lax.fori_loop(0, tw // ob, block, 0)

    return sck


def _sc_gather_reduce(a, gidx2):
    nb, _, c = a.shape
    bn = nb * N
    smax2, ssum2 = _make_sc_gather_reduce(c, bn)(a.reshape(bn, c), gidx2)
    return smax2.reshape(nb, N, c), ssum2.reshape(nb, N, c)


def _stats_body(c, a_ref, d_ref, mx_ref, sm_ref, cnt_ref, gw_ref, gb_ref, o_ref):
    cg = c // 4
    a = a_ref[0]
    d = d_ref[0]
    mx = mx_ref[0].astype(jnp.float32)
    sm = sm_ref[0].astype(jnp.float32)
    cnt = cnt_ref[0]
    col_a = jnp.sum(a * cnt, axis=0, keepdims=True)
    col_a2 = jnp.sum(a * a * cnt, axis=0, keepdims=True)
    col_d = jnp.sum(d, axis=0, keepdims=True)
    col_d2 = jnp.sum(d * d, axis=0, keepdims=True)
    col_x = jnp.sum(d * sm, axis=0, keepdims=True)
    s1 = col_a + K * col_d
    s2 = col_a2 + 2.0 * col_x + K * col_d2
    ci = lax.broadcasted_iota(jnp.int32, (1, c), 1)
    denom = float(cg * N * K)
    mean_bc = jnp.zeros((1, c), jnp.float32)
    inv_bc = jnp.zeros((1, c), jnp.float32)
    for g in range(4):
        msk = (ci >= g * cg) & (ci < (g + 1) * cg)
        zero = jnp.zeros((1, c), jnp.float32)
        s1g = jnp.sum(jnp.where(msk, s1, zero), axis=1, keepdims=True)
        s2g = jnp.sum(jnp.where(msk, s2, zero), axis=1, keepdims=True)
        mean_g = s1g / denom
        var_g = s2g / denom - mean_g * mean_g
        inv_g = lax.rsqrt(var_g + EPS)
        mean_bc = jnp.where(msk, mean_bc + mean_g, mean_bc)
        inv_bc = jnp.where(msk, inv_bc + inv_g, inv_bc)
    xn = (d + mx - mean_bc) * inv_bc * gw_ref[...] + gb_ref[...]
    o_ref[0] = _leaky(xn)


def _stats(a, d, smax, ssum, cnt3, gnw, gnb):
    c = a.shape[-1]
    nb = a.shape[0]
    spec = pl.BlockSpec((1, N, c), lambda b: (b, 0, 0))
    wspec = pl.BlockSpec((1, c), lambda b: (0, 0))
    return pl.pallas_call(
        functools.partial(_stats_body, c),
        grid=(nb,),
        in_specs=[spec, spec, spec, spec,
                  pl.BlockSpec((1, N, 1), lambda b: (b, 0, 0)),
                  wspec, wspec],
        out_specs=spec,
        out_shape=jax.ShapeDtypeStruct((nb, N, c), jnp.float32),
    )(a, d, smax, ssum, cnt3, gnw.reshape(1, c), gnb.reshape(1, c))


def _final_body(f1_ref, f2_ref, f3_ref, f4_ref, w_ref, gw_ref, gb_ref, o_ref):
    outc = w_ref.shape[0]
    cg = outc // 4
    p = _dot_t(f1_ref[0], w_ref[:, 0:256])
    p = p + _dot_t(f2_ref[0], w_ref[:, 256:768])
    p = p + _dot_t(f3_ref[0], w_ref[:, 768:1280])
    p = p + _dot_t(f4_ref[0], w_ref[:, 1280:2304])
    s1 = jnp.sum(p, axis=0, keepdims=True)
    s2 = jnp.sum(p * p, axis=0, keepdims=True)
    ci = lax.broadcasted_iota(jnp.int32, (1, outc), 1)
    denom = float(cg * N)
    mean_bc = jnp.zeros((1, outc), jnp.float32)
    inv_bc = jnp.zeros((1, outc), jnp.float32)
    for g in range(4):
        msk = (ci >= g * cg) & (ci < (g + 1) * cg)
        zero = jnp.zeros((1, outc), jnp.float32)
        s1g = jnp.sum(jnp.where(msk, s1, zero), axis=1, keepdims=True)
        s2g = jnp.sum(jnp.where(msk, s2, zero), axis=1, keepdims=True)
        mean_g = s1g / denom
        var_g = s2g / denom - mean_g * mean_g
        inv_g = lax.rsqrt(var_g + EPS)
        mean_bc = jnp.where(msk, mean_bc + mean_g, mean_bc)
        inv_bc = jnp.where(msk, inv_bc + inv_g, inv_bc)
    xn = (p - mean_bc) * inv_bc * gw_ref[...] + gb_ref[...]
    o_ref[0] = _leaky(xn)


def _final(f1, f2, f3, f4, w5, gw, gb):
    outc = w5.shape[0]
    nb = f1.shape[0]
    return pl.pallas_call(
        _final_body,
        grid=(nb,),
        in_specs=[
            pl.BlockSpec((1, N, 256), lambda b: (b, 0, 0)),
            pl.BlockSpec((1, N, 512), lambda b: (b, 0, 0)),
            pl.BlockSpec((1, N, 512), lambda b: (b, 0, 0)),
            pl.BlockSpec((1, N, 1024), lambda b: (b, 0, 0)),
            pl.BlockSpec((outc, 2304), lambda b: (0, 0)),
            pl.BlockSpec((1, outc), lambda b: (0, 0)),
            pl.BlockSpec((1, outc), lambda b: (0, 0)),
        ],
        out_specs=pl.BlockSpec((1, N, outc), lambda b: (b, 0, 0)),
        out_shape=jax.ShapeDtypeStruct((nb, N, outc), jnp.float32),
    )(f1, f2, f3, f4, w5, gw.reshape(1, outc), gb.reshape(1, outc))


def kernel(f, coor, W_in, b_in, W1, gn1_w, gn1_b, W2, gn2_w, gn2_b,
           W3, gn3_w, gn3_b, W4, gn4_w, gn4_b, W5, gn5_w, gn5_b):
    gidx, cnt = _knn(coor)
    cnt3 = jnp.transpose(cnt, (0, 2, 1))
    f0 = _intrans(f, W_in, b_in)

    layers = ((W1, gn1_w, gn1_b), (W2, gn2_w, gn2_b),
              (W3, gn3_w, gn3_b), (W4, gn4_w, gn4_b))
    fcur = [f0[b:b + 1] for b in range(B)]
    gidx_b = [gidx[b].reshape(N, K) for b in range(B)]
    cnt_b = [cnt3[b:b + 1] for b in range(B)]
    feats = [[] for _ in range(B)]
    for (w, gnw, gnb) in layers:
        ad = [_mm(fcur[b], w) for b in range(B)]
        sc = [_sc_gather_reduce(ad[b][0], gidx_b[b]) for b in range(B)]
        for b in range(B):
            a, d = ad[b]
            smax, ssum = sc[b]
            fcur[b] = _stats(a, d, smax, ssum, cnt_b[b], gnw, gnb)
            feats[b].append(fcur[b])
    outs = [_final(feats[b][0], feats[b][1], feats[b][2], feats[b][3],
                   W5, gn5_w, gn5_b) for b in range(B)]
    return jnp.concatenate(outs, axis=0)
